# baseline (speedup 1.0000x reference)
import functools

import jax
import jax.numpy as jnp
from jax import lax
from jax.experimental import pallas as pl
from jax.experimental.pallas import tpu as pltpu
from jax.experimental.pallas import tpu_sc as plsc

N = 10000
NPAD = 10240
E = 320000
G = 120
N_TILES = 32
NWIN = 84
EPT = G * NWIN
EP = EPT * N_TILES
RPT = NPAD // 16
CHUNKS = (120, 120, 120, 120, 120, 40)
EPS = 1e-5


def _make_sc_agg(F):
    mesh = plsc.VectorSubcoreMesh(core_axis_name="c", subcore_axis_name="s")

    @functools.partial(
        pl.kernel,
        out_type=jax.ShapeDtypeStruct((2, NPAD, F), jnp.float32),
        mesh=mesh,
        scratch_types=(
            [pltpu.VMEM((G,), jnp.int32) for _ in range(6)]
            + [pltpu.VMEM((G,), jnp.int32) for _ in range(6)]
            + [pltpu.VMEM((G, F), jnp.float32) for _ in range(3)]
            + [pltpu.VMEM_SHARED((NPAD, F), jnp.float32)]
            + [pltpu.SemaphoreType.DMA for _ in range(12)]
        ),
    )
    def agg(h_hbm, src_hbm, tgt_hbm, zeros_hbm, out_hbm, *rest):
        src_b = rest[0:6]
        tgt_b = rest[6:12]
        rows_v = rest[12:15]
        acc_sh = rest[15]
        sem_i = rest[16:22]
        sem_g = rest[22:25]
        sem_s = rest[25:28]

        c = lax.axis_index("c")
        s = lax.axis_index("s")
        wid = c * 16 + s
        ebase = wid * EPT

        pltpu.sync_copy(zeros_hbm, rows_v[0])
        off = 0
        for ch in CHUNKS:
            pltpu.sync_copy(rows_v[0].at[pl.ds(0, ch)],
                            acc_sh.at[pl.ds(s * RPT + off, ch)])
            off += ch
        plsc.subcore_barrier()

        def idx_cps(ww, q):
            base = ebase + ww * G
            return [
                pltpu.make_async_copy(src_hbm.at[pl.ds(base, G)], src_b[q],
                                      sem_i[q]),
                pltpu.make_async_copy(tgt_hbm.at[pl.ds(base, G)], tgt_b[q],
                                      sem_i[q]),
            ]

        def win_block(ww, b, drain, guard_prefetch, lagged_scatter):
            rb = b % 3
            q = b % 6
            q3 = (b + 3) % 6
            rbm1 = (b + 2) % 3
            qm1 = (b + 5) % 6
            if drain:
                pltpu.make_async_copy(rows_v[rb], acc_sh.at[tgt_b[q3]],
                                      sem_s[rb]).wait()
            if guard_prefetch:
                @pl.when(ww + 3 < NWIN)
                def _():
                    for cp in idx_cps(ww + 3, q3):
                        cp.start()
            else:
                for cp in idx_cps(ww + 3, q3):
                    cp.start()
            for cp in idx_cps(ww, q):
                cp.wait()
            pltpu.async_copy(h_hbm.at[src_b[q]], rows_v[rb], sem_g[rb])
            if lagged_scatter:
                pltpu.make_async_copy(h_hbm.at[src_b[qm1]], rows_v[rbm1],
                                      sem_g[rbm1]).wait()
                pltpu.async_copy(rows_v[rbm1], acc_sh.at[tgt_b[qm1]],
                                 sem_s[rbm1], add=True)

        for q in range(3):
            for cp in idx_cps(q, q):
                cp.start()
        for b in range(6):
            win_block(b, b, drain=(b >= 3), guard_prefetch=False,
                      lagged_scatter=(b >= 1))

        @pl.loop(6, NWIN, step=6)
        def _(w):
            for b in range(6):
                win_block(w + b, b, drain=True, guard_prefetch=True,
                          lagged_scatter=True)

        lb = (NWIN - 1) % 6
        pltpu.make_async_copy(h_hbm.at[src_b[lb % 6]], rows_v[lb % 3],
                              sem_g[lb % 3]).wait()
        pltpu.async_copy(rows_v[lb % 3], acc_sh.at[tgt_b[lb % 6]],
                         sem_s[lb % 3], add=True)
        for ww in (NWIN - 3, NWIN - 2, NWIN - 1):
            pltpu.make_async_copy(rows_v[ww % 3], acc_sh.at[tgt_b[ww % 6]],
                                  sem_s[ww % 3]).wait()
        plsc.subcore_barrier()

        nch = len(CHUNKS)
        ins = []
        off = 0
        for ch in CHUNKS:
            start = s * RPT + off
            ins.append((ch, acc_sh.at[pl.ds(start, ch)],
                        out_hbm.at[c].at[pl.ds(start, ch)]))
            off += ch
        pltpu.async_copy(ins[0][1], rows_v[0].at[pl.ds(0, ins[0][0])],
                         sem_g[0])
        for k in range(nch):
            rb = k % 2
            rb1 = (k + 1) % 2
            pltpu.make_async_copy(ins[k][1],
                                  rows_v[rb].at[pl.ds(0, ins[k][0])],
                                  sem_g[rb]).wait()
            if k + 1 < nch:
                if k >= 1:
                    pltpu.make_async_copy(
                        rows_v[rb1].at[pl.ds(0, ins[k - 1][0])],
                        ins[k - 1][2], sem_s[rb1]).wait()
                pltpu.async_copy(ins[k + 1][1],
                                 rows_v[rb1].at[pl.ds(0, ins[k + 1][0])],
                                 sem_g[rb1])
            pltpu.async_copy(rows_v[rb].at[pl.ds(0, ins[k][0])], ins[k][2],
                             sem_s[rb])
        for k in (nch - 2, nch - 1):
            pltpu.make_async_copy(rows_v[k % 2].at[pl.ds(0, ins[k][0])],
                                  ins[k][2], sem_s[k % 2]).wait()

    return agg


_sc_agg_128 = _make_sc_agg(128)


BR = 10000


def _row_spec(shape_f):
    return pl.BlockSpec((BR,) + shape_f, lambda i: (i,) + (0,) * len(shape_f))


def _full_spec(shape):
    return pl.BlockSpec(shape, lambda i: (0,) * len(shape))


def _stage_a_body(x_ref, w_ref, b_ref, t_ref):
    t_ref[...] = jnp.dot(x_ref[...], w_ref[...],
                         preferred_element_type=jnp.float32) + b_ref[...]


def _stage_a(x, W, b):
    F = W.shape[1]
    return pl.pallas_call(
        _stage_a_body,
        grid=(N // BR,),
        in_specs=[_row_spec((128,)), _full_spec((128, F)), _full_spec((1, F))],
        out_specs=_row_spec((F,)),
        out_shape=jax.ShapeDtypeStruct((N, F), jnp.float32),
    )(x, W, b.reshape(1, F))


def _stage_b_body(p_ref, m_ref, w_ref, b_ref, h_ref, t_ref):
    h = jax.nn.relu(m_ref[...] * (p_ref[0] + p_ref[1]))
    h_ref[...] = h
    t_ref[...] = jnp.dot(h, w_ref[...],
                         preferred_element_type=jnp.float32) + b_ref[...]


def _stage_b(p, Mtgt, W, b):
    F = W.shape[1]
    return pl.pallas_call(
        _stage_b_body,
        grid=(N // BR,),
        in_specs=[
            pl.BlockSpec((2, BR, 128), lambda i: (0, i, 0)),
            _row_spec((1,)),
            _full_spec((128, F)),
            _full_spec((1, F)),
        ],
        out_specs=[_row_spec((128,)), _row_spec((F,))],
        out_shape=[
            jax.ShapeDtypeStruct((N, 128), jnp.float32),
            jax.ShapeDtypeStruct((N, F), jnp.float32),
        ],
    )(p, Mtgt, W, b.reshape(1, F))


def _stage_mid_body(p_ref, m_ref, g_ref, gw_ref, gb_ref, r_ref, w_ref, b_ref,
                    h_ref, t_ref):
    z = jax.nn.relu(m_ref[...] * (p_ref[0] + p_ref[1]))
    mean = jnp.dot(z, g_ref[...], preferred_element_type=jnp.float32)
    d = z - mean
    var = jnp.dot(d * d, g_ref[...], preferred_element_type=jnp.float32)
    gn = d * lax.rsqrt(var + EPS) * gw_ref[...] + gb_ref[...]
    h = gn + r_ref[...]
    h_ref[...] = h
    t_ref[...] = jnp.dot(h, w_ref[...],
                         preferred_element_type=jnp.float32) + b_ref[...]


def _stage_mid(p, Mtgt, Gmat, gw, gb, resid, W, b):
    F = W.shape[1]
    return pl.pallas_call(
        _stage_mid_body,
        grid=(N // BR,),
        in_specs=[
            pl.BlockSpec((2, BR, 128), lambda i: (0, i, 0)),
            _row_spec((1,)),
            _full_spec((128, 128)),
            _full_spec((1, 128)),
            _full_spec((1, 128)),
            _row_spec((128,)),
            _full_spec((128, F)),
            _full_spec((1, F)),
        ],
        out_specs=[_row_spec((128,)), _row_spec((F,))],
        out_shape=[
            jax.ShapeDtypeStruct((N, 128), jnp.float32),
            jax.ShapeDtypeStruct((N, F), jnp.float32),
        ],
    )(p, Mtgt, Gmat, gw.reshape(1, 128), gb.reshape(1, 128), resid, W,
      b.reshape(1, F))


def _stage_e_body(p_ref, m_ref, o_ref):
    C = o_ref.shape[1]
    o = (m_ref[...] * (p_ref[0] + p_ref[1]))[:, :C]
    mx = jnp.max(o, axis=1, keepdims=True)
    lse = jnp.log(jnp.sum(jnp.exp(o - mx), axis=1, keepdims=True)) + mx
    o_ref[...] = o - lse


def _stage_e(p, Mtgt, C):
    return pl.pallas_call(
        _stage_e_body,
        grid=(N // BR,),
        in_specs=[
            pl.BlockSpec((2, BR, 128), lambda i: (0, i, 0)),
            _row_spec((1,)),
        ],
        out_specs=_row_spec((C,)),
        out_shape=jax.ShapeDtypeStruct((N, C), jnp.float32),
    )(p, Mtgt)


def kernel(x, src, tgt, Mtgt, W0, b0, W1, b1, W2, b2, W3, b3,
           g1w, g1b, g2w, g2b):
    zeros = jnp.zeros((G, 128), jnp.float32)
    Gmat = jnp.kron(jnp.eye(32, dtype=jnp.float32),
                    jnp.full((4, 4), 0.25, jnp.float32))
    nclass = W3.shape[1]
    W3p = jnp.pad(W3, ((0, 0), (0, 128 - nclass)))
    b3p = jnp.pad(b3, (0, 128 - nclass))
    pad = EP - E
    src = jnp.concatenate([src, (jnp.arange(pad, dtype=jnp.int32) * 131) % N])
    tgt = jnp.concatenate(
        [tgt, N + jnp.arange(pad, dtype=jnp.int32) % (NPAD - N)])

    t0 = _stage_a(x, W0, b0)
    p0 = _sc_agg_128(t0, src, tgt, zeros)
    h0, t1 = _stage_b(p0, Mtgt, W1, b1)
    p1 = _sc_agg_128(t1, src, tgt, zeros)
    h1, t2 = _stage_mid(p1, Mtgt, Gmat, g1w, g1b, h0, W2, b2)
    p2 = _sc_agg_128(t2, src, tgt, zeros)
    _, t3 = _stage_mid(p2, Mtgt, Gmat, g2w, g2b, h1, W3p, b3p)
    p3 = _sc_agg_128(t3, src, tgt, zeros)
    return _stage_e(p3, Mtgt, nclass)


# speedup vs baseline: 1.0214x; 1.0214x over previous
import functools

import jax
import jax.numpy as jnp
from jax import lax
from jax.experimental import pallas as pl
from jax.experimental.pallas import tpu as pltpu
from jax.experimental.pallas import tpu_sc as plsc

N = 10000
NPAD = 10240
E = 320000
G = 120
N_TILES = 32
NWIN = 84
EPT = G * NWIN
EP = EPT * N_TILES
RPT = NPAD // 16
CHUNKS = (120, 120, 120, 120, 120, 40)
EPS = 1e-5


def _make_sc_agg(F):
    mesh = plsc.VectorSubcoreMesh(core_axis_name="c", subcore_axis_name="s")

    @functools.partial(
        pl.kernel,
        out_type=jax.ShapeDtypeStruct((2, NPAD, F), jnp.float32),
        mesh=mesh,
        scratch_types=(
            [pltpu.VMEM((G,), jnp.int32) for _ in range(6)]
            + [pltpu.VMEM((G,), jnp.int32) for _ in range(6)]
            + [pltpu.VMEM((G, F), jnp.float32) for _ in range(3)]
            + [pltpu.VMEM_SHARED((NPAD, F), jnp.float32)]
            + [pltpu.SemaphoreType.DMA for _ in range(12)]
        ),
    )
    def agg(h_hbm, src_hbm, tgt_hbm, zeros_hbm, out_hbm, *rest):
        src_b = rest[0:6]
        tgt_b = rest[6:12]
        rows_v = rest[12:15]
        acc_sh = rest[15]
        sem_i = rest[16:22]
        sem_g = rest[22:25]
        sem_s = rest[25:28]

        c = lax.axis_index("c")
        s = lax.axis_index("s")
        wid = c * 16 + s
        ebase = wid * EPT

        pltpu.sync_copy(zeros_hbm, rows_v[0])
        off = 0
        for ch in CHUNKS:
            pltpu.sync_copy(rows_v[0].at[pl.ds(0, ch)],
                            acc_sh.at[pl.ds(s * RPT + off, ch)])
            off += ch
        plsc.subcore_barrier()

        def idx_cps(ww, q):
            base = ebase + ww * G
            return [
                pltpu.make_async_copy(src_hbm.at[pl.ds(base, G)], src_b[q],
                                      sem_i[q]),
                pltpu.make_async_copy(tgt_hbm.at[pl.ds(base, G)], tgt_b[q],
                                      sem_i[q]),
            ]

        def win_block(ww, b, drain, guard_prefetch, lagged_scatter):
            rb = b % 3
            q = b % 6
            q3 = (b + 3) % 6
            rbm1 = (b + 2) % 3
            qm1 = (b + 5) % 6
            if drain:
                pltpu.make_async_copy(rows_v[rb], acc_sh.at[tgt_b[q3]],
                                      sem_s[rb]).wait()
            if guard_prefetch:
                @pl.when(ww + 3 < NWIN)
                def _():
                    for cp in idx_cps(ww + 3, q3):
                        cp.start()
            else:
                for cp in idx_cps(ww + 3, q3):
                    cp.start()
            for cp in idx_cps(ww, q):
                cp.wait()
            pltpu.async_copy(h_hbm.at[src_b[q]], rows_v[rb], sem_g[rb])
            if lagged_scatter:
                pltpu.make_async_copy(h_hbm.at[src_b[qm1]], rows_v[rbm1],
                                      sem_g[rbm1]).wait()
                pltpu.async_copy(rows_v[rbm1], acc_sh.at[tgt_b[qm1]],
                                 sem_s[rbm1], add=True)

        for q in range(3):
            for cp in idx_cps(q, q):
                cp.start()
        for b in range(6):
            win_block(b, b, drain=(b >= 3), guard_prefetch=False,
                      lagged_scatter=(b >= 1))

        @pl.loop(6, NWIN, step=6)
        def _(w):
            for b in range(6):
                win_block(w + b, b, drain=True, guard_prefetch=True,
                          lagged_scatter=True)

        lb = (NWIN - 1) % 6
        pltpu.make_async_copy(h_hbm.at[src_b[lb % 6]], rows_v[lb % 3],
                              sem_g[lb % 3]).wait()
        pltpu.async_copy(rows_v[lb % 3], acc_sh.at[tgt_b[lb % 6]],
                         sem_s[lb % 3], add=True)
        for ww in (NWIN - 3, NWIN - 2, NWIN - 1):
            pltpu.make_async_copy(rows_v[ww % 3], acc_sh.at[tgt_b[ww % 6]],
                                  sem_s[ww % 3]).wait()
        plsc.subcore_barrier()

        nch = len(CHUNKS)
        ins = []
        off = 0
        for ch in CHUNKS:
            start = s * RPT + off
            ins.append((ch, acc_sh.at[pl.ds(start, ch)],
                        out_hbm.at[c].at[pl.ds(start, ch)]))
            off += ch
        pltpu.async_copy(ins[0][1], rows_v[0].at[pl.ds(0, ins[0][0])],
                         sem_g[0])
        for k in range(nch):
            rb = k % 2
            rb1 = (k + 1) % 2
            pltpu.make_async_copy(ins[k][1],
                                  rows_v[rb].at[pl.ds(0, ins[k][0])],
                                  sem_g[rb]).wait()
            if k + 1 < nch:
                if k >= 1:
                    pltpu.make_async_copy(
                        rows_v[rb1].at[pl.ds(0, ins[k - 1][0])],
                        ins[k - 1][2], sem_s[rb1]).wait()
                pltpu.async_copy(ins[k + 1][1],
                                 rows_v[rb1].at[pl.ds(0, ins[k + 1][0])],
                                 sem_g[rb1])
            pltpu.async_copy(rows_v[rb].at[pl.ds(0, ins[k][0])], ins[k][2],
                             sem_s[rb])
        for k in (nch - 2, nch - 1):
            pltpu.make_async_copy(rows_v[k % 2].at[pl.ds(0, ins[k][0])],
                                  ins[k][2], sem_s[k % 2]).wait()

    return agg


_sc_agg_128 = _make_sc_agg(128)


BR = 5000


def _row_spec(shape_f):
    return pl.BlockSpec((BR,) + shape_f, lambda i: (i,) + (0,) * len(shape_f))


def _full_spec(shape):
    return pl.BlockSpec(shape, lambda i: (0,) * len(shape))


def _stage_a_body(x_ref, w_ref, b_ref, t_ref):
    t_ref[...] = jnp.dot(x_ref[...], w_ref[...],
                         preferred_element_type=jnp.float32) + b_ref[...]


def _stage_a(x, W, b):
    F = W.shape[1]
    return pl.pallas_call(
        _stage_a_body,
        grid=(N // BR,),
        in_specs=[_row_spec((128,)), _full_spec((128, F)), _full_spec((1, F))],
        out_specs=_row_spec((F,)),
        out_shape=jax.ShapeDtypeStruct((N, F), jnp.float32),
    )(x, W, b.reshape(1, F))


def _stage_b_body(p_ref, m_ref, w_ref, b_ref, h_ref, t_ref):
    h = jax.nn.relu(m_ref[...] * (p_ref[0] + p_ref[1]))
    h_ref[...] = h
    t_ref[...] = jnp.dot(h, w_ref[...],
                         preferred_element_type=jnp.float32) + b_ref[...]


def _stage_b(p, Mtgt, W, b):
    F = W.shape[1]
    return pl.pallas_call(
        _stage_b_body,
        grid=(N // BR,),
        in_specs=[
            pl.BlockSpec((2, BR, 128), lambda i: (0, i, 0)),
            _row_spec((1,)),
            _full_spec((128, F)),
            _full_spec((1, F)),
        ],
        out_specs=[_row_spec((128,)), _row_spec((F,))],
        out_shape=[
            jax.ShapeDtypeStruct((N, 128), jnp.float32),
            jax.ShapeDtypeStruct((N, F), jnp.float32),
        ],
    )(p, Mtgt, W, b.reshape(1, F))


def _stage_mid_body(p_ref, m_ref, g_ref, gw_ref, gb_ref, r_ref, w_ref, b_ref,
                    h_ref, t_ref):
    z = jax.nn.relu(m_ref[...] * (p_ref[0] + p_ref[1]))
    mean = jnp.dot(z, g_ref[...], preferred_element_type=jnp.float32)
    d = z - mean
    var = jnp.dot(d * d, g_ref[...], preferred_element_type=jnp.float32)
    gn = d * lax.rsqrt(var + EPS) * gw_ref[...] + gb_ref[...]
    h = gn + r_ref[...]
    h_ref[...] = h
    t_ref[...] = jnp.dot(h, w_ref[...],
                         preferred_element_type=jnp.float32) + b_ref[...]


def _stage_mid(p, Mtgt, Gmat, gw, gb, resid, W, b):
    F = W.shape[1]
    return pl.pallas_call(
        _stage_mid_body,
        grid=(N // BR,),
        in_specs=[
            pl.BlockSpec((2, BR, 128), lambda i: (0, i, 0)),
            _row_spec((1,)),
            _full_spec((128, 128)),
            _full_spec((1, 128)),
            _full_spec((1, 128)),
            _row_spec((128,)),
            _full_spec((128, F)),
            _full_spec((1, F)),
        ],
        out_specs=[_row_spec((128,)), _row_spec((F,))],
        out_shape=[
            jax.ShapeDtypeStruct((N, 128), jnp.float32),
            jax.ShapeDtypeStruct((N, F), jnp.float32),
        ],
    )(p, Mtgt, Gmat, gw.reshape(1, 128), gb.reshape(1, 128), resid, W,
      b.reshape(1, F))


def _stage_e_body(p_ref, m_ref, o_ref):
    C = o_ref.shape[1]
    o = (m_ref[...] * (p_ref[0] + p_ref[1]))[:, :C]
    mx = jnp.max(o, axis=1, keepdims=True)
    lse = jnp.log(jnp.sum(jnp.exp(o - mx), axis=1, keepdims=True)) + mx
    o_ref[...] = o - lse


def _stage_e(p, Mtgt, C):
    return pl.pallas_call(
        _stage_e_body,
        grid=(N // BR,),
        in_specs=[
            pl.BlockSpec((2, BR, 128), lambda i: (0, i, 0)),
            _row_spec((1,)),
        ],
        out_specs=_row_spec((C,)),
        out_shape=jax.ShapeDtypeStruct((N, C), jnp.float32),
    )(p, Mtgt)


def kernel(x, src, tgt, Mtgt, W0, b0, W1, b1, W2, b2, W3, b3,
           g1w, g1b, g2w, g2b):
    zeros = jnp.zeros((G, 128), jnp.float32)
    Gmat = jnp.kron(jnp.eye(32, dtype=jnp.float32),
                    jnp.full((4, 4), 0.25, jnp.float32))
    nclass = W3.shape[1]
    W3p = jnp.pad(W3, ((0, 0), (0, 128 - nclass)))
    b3p = jnp.pad(b3, (0, 128 - nclass))
    pad = EP - E
    src = jnp.concatenate([src, (jnp.arange(pad, dtype=jnp.int32) * 131) % N])
    tgt = jnp.concatenate(
        [tgt, N + jnp.arange(pad, dtype=jnp.int32) % (NPAD - N)])

    t0 = _stage_a(x, W0, b0)
    p0 = _sc_agg_128(t0, src, tgt, zeros)
    h0, t1 = _stage_b(p0, Mtgt, W1, b1)
    p1 = _sc_agg_128(t1, src, tgt, zeros)
    h1, t2 = _stage_mid(p1, Mtgt, Gmat, g1w, g1b, h0, W2, b2)
    p2 = _sc_agg_128(t2, src, tgt, zeros)
    _, t3 = _stage_mid(p2, Mtgt, Gmat, g2w, g2b, h1, W3p, b3p)
    p3 = _sc_agg_128(t3, src, tgt, zeros)
    return _stage_e(p3, Mtgt, nclass)
